# baseline (device time: 29616 ns/iter reference)
import jax
import jax.numpy as jnp
from jax import lax
from jax.experimental import pallas as pl
from jax.experimental.pallas import tpu as pltpu

N_DEV = 4
K = 16
CHUNK = 128
N_QUARTERS = 4
NEG = float("-inf")

_USE_TPU_ROLL = True


def _bitonic_sort_desc(L):
    L = list(L)
    k = len(L)
    d = k // 2
    while d >= 1:
        for blk in range(0, k, 2 * d):
            for i in range(blk, blk + d):
                hi = jnp.maximum(L[i], L[i + d])
                lo = jnp.minimum(L[i], L[i + d])
                L[i], L[i + d] = hi, lo
        d //= 2
    return L


def _merge_desc(A, B):
    return _bitonic_sort_desc(A + B[::-1])


def _merge_topk_desc(A, B):
    k = len(A)
    top = [jnp.maximum(A[i], B[k - 1 - i]) for i in range(k)]
    return _bitonic_sort_desc(top)


def _tree_merge(lists):
    while len(lists) > 1:
        nxt = []
        for j in range(0, len(lists), 2):
            A, B = lists[j], lists[j + 1]
            if len(A) >= K:
                nxt.append(_merge_topk_desc(A, B))
            else:
                nxt.append(_merge_desc(A, B))
        lists = nxt
    return lists[0]


def _lroll(a, s):
    if _USE_TPU_ROLL:
        return pltpu.roll(a, a.shape[1] - s, axis=1)
    return jnp.roll(a, -s, axis=1)


def _topk_extract_desc(work, k):
    out_cols = []
    for j in range(k):
        m = jnp.max(work, axis=1, keepdims=True)
        out_cols.append(m)
        if j < k - 1:
            work = jnp.where(work == m, NEG, work)
    return jnp.concatenate(out_cols, axis=1)


def _finish_topk(R):
    w = CHUNK
    while w > K:
        h = w // 2
        top = [jnp.maximum(R[i], _lroll(R[K - 1 - i], h)) for i in range(K)]
        R = top if h == K else _bitonic_sort_desc(top)
        w = h
    cand = jnp.concatenate([r[:, :K] for r in R], axis=1)
    return _topk_extract_desc(cand, K)


def _local_topk_from_chunk_lists(quarter_lists):
    R = _tree_merge(quarter_lists)
    return _finish_topk(R)


def kernel(x):
    m, n = x.shape
    q_cols = n // N_QUARTERS
    chunks_per_q = q_cols // CHUNK

    def body(x_hbm, out_ref, xq_ref, cand_ref, load_sems, send_sems,
             recv_sems):
        my_pos = lax.axis_index("i")

        barrier_sem = pltpu.get_barrier_semaphore()
        for o in range(1, N_DEV):
            pl.semaphore_signal(
                barrier_sem, inc=1,
                device_id=(lax.rem(my_pos + o, N_DEV),),
                device_id_type=pl.DeviceIdType.MESH,
            )

        copies = []
        for q in range(N_QUARTERS):
            cp = pltpu.make_async_copy(
                x_hbm.at[:, q * q_cols:(q + 1) * q_cols],
                xq_ref.at[q],
                load_sems.at[q],
            )
            cp.start()
            copies.append(cp)

        quarter_lists = []
        for q in range(N_QUARTERS):
            copies[q].wait()
            chunks = [
                xq_ref[q, :, c * CHUNK:(c + 1) * CHUNK]
                for c in range(chunks_per_q)
            ]
            quarter_lists.append(_tree_merge([[c] for c in chunks]))

        local = _local_topk_from_chunk_lists(quarter_lists)
        cand_ref[pl.ds(my_pos, 1)] = local[None, :, :]

        pl.semaphore_wait(barrier_sem, N_DEV - 1)

        rdmas = []
        for o in range(1, N_DEV):
            peer = lax.rem(my_pos + o, N_DEV)
            rdma = pltpu.make_async_remote_copy(
                src_ref=cand_ref.at[my_pos],
                dst_ref=cand_ref.at[my_pos],
                send_sem=send_sems.at[o - 1],
                recv_sem=recv_sems.at[o - 1],
                device_id=(peer,),
                device_id_type=pl.DeviceIdType.MESH,
            )
            rdma.start()
            rdmas.append(rdma)
        for rdma in rdmas:
            rdma.wait()

        gathered = jnp.concatenate(
            [cand_ref[i] for i in range(N_DEV)], axis=1
        )
        out_ref[:, :] = _topk_extract_desc(gathered, K)

    return pl.pallas_call(
        body,
        out_shape=jax.ShapeDtypeStruct((m, K), jnp.float32),
        in_specs=[pl.BlockSpec(memory_space=pl.ANY)],
        out_specs=pl.BlockSpec(memory_space=pltpu.VMEM),
        scratch_shapes=[
            pltpu.VMEM((N_QUARTERS, m, q_cols), jnp.float32),
            pltpu.VMEM((N_DEV, m, K), jnp.float32),
            pltpu.SemaphoreType.DMA((N_QUARTERS,)),
            pltpu.SemaphoreType.DMA((N_DEV - 1,)),
            pltpu.SemaphoreType.DMA((N_DEV - 1,)),
        ],
        compiler_params=pltpu.CompilerParams(collective_id=0),
    )(x)


# device time: 25409 ns/iter; 1.1656x vs baseline; 1.1656x over previous
import jax
import jax.numpy as jnp
from jax import lax
from jax.experimental import pallas as pl
from jax.experimental.pallas import tpu as pltpu

N_DEV = 4
K = 16
CHUNK = 128
N_QUARTERS = 4
NEG = float("-inf")

_USE_TPU_ROLL = True


def _bitonic_sort_desc(L):
    L = list(L)
    k = len(L)
    d = k // 2
    while d >= 1:
        for blk in range(0, k, 2 * d):
            for i in range(blk, blk + d):
                hi = jnp.maximum(L[i], L[i + d])
                lo = jnp.minimum(L[i], L[i + d])
                L[i], L[i + d] = hi, lo
        d //= 2
    return L


def _merge_desc(A, B):
    return _bitonic_sort_desc(A + B[::-1])


def _merge_topk_desc(A, B):
    k = len(A)
    top = [jnp.maximum(A[i], B[k - 1 - i]) for i in range(k)]
    return _bitonic_sort_desc(top)


def _tree_merge(lists):
    while len(lists) > 1:
        nxt = []
        for j in range(0, len(lists), 2):
            A, B = lists[j], lists[j + 1]
            if len(A) >= K:
                nxt.append(_merge_topk_desc(A, B))
            else:
                nxt.append(_merge_desc(A, B))
        lists = nxt
    return lists[0]


def _lroll(a, s):
    if _USE_TPU_ROLL:
        return pltpu.roll(a, a.shape[1] - s, axis=1)
    return jnp.roll(a, -s, axis=1)


def _topk_extract_desc(work, k):
    out_cols = []
    for j in range(k):
        m = jnp.max(work, axis=1, keepdims=True)
        out_cols.append(m)
        if j < k - 1:
            work = jnp.where(work == m, NEG, work)
    return jnp.concatenate(out_cols, axis=1)


def _finish_topk(R):
    w = CHUNK
    while w > K:
        h = w // 2
        top = [jnp.maximum(R[i], _lroll(R[K - 1 - i], h)) for i in range(K)]
        R = top if h == K else _bitonic_sort_desc(top)
        w = h
    cand = jnp.concatenate([r[:, :K] for r in R], axis=1)
    return _topk_extract_desc(cand, K)


def _local_topk_from_chunk_lists(quarter_lists):
    R = _tree_merge(quarter_lists)
    return _finish_topk(R)


def kernel(x):
    m, n = x.shape
    q_cols = n // N_QUARTERS
    chunks_per_q = q_cols // CHUNK

    def body(x_ref, out_ref, cand_ref, send_sems, recv_sems):
        my_pos = lax.axis_index("i")

        barrier_sem = pltpu.get_barrier_semaphore()
        for o in range(1, N_DEV):
            pl.semaphore_signal(
                barrier_sem, inc=1,
                device_id=(lax.rem(my_pos + o, N_DEV),),
                device_id_type=pl.DeviceIdType.MESH,
            )

        chunks = [
            x_ref[:, c * CHUNK:(c + 1) * CHUNK] for c in range(n // CHUNK)
        ]
        local = _local_topk_from_chunk_lists([[c] for c in chunks])
        cand_ref[pl.ds(my_pos, 1)] = local[None, :, :]

        pl.semaphore_wait(barrier_sem, N_DEV - 1)

        rdmas = []
        for o in range(1, N_DEV):
            peer = lax.rem(my_pos + o, N_DEV)
            rdma = pltpu.make_async_remote_copy(
                src_ref=cand_ref.at[my_pos],
                dst_ref=cand_ref.at[my_pos],
                send_sem=send_sems.at[o - 1],
                recv_sem=recv_sems.at[o - 1],
                device_id=(peer,),
                device_id_type=pl.DeviceIdType.MESH,
            )
            rdma.start()
            rdmas.append(rdma)
        for rdma in rdmas:
            rdma.wait()

        gathered = jnp.concatenate(
            [cand_ref[i] for i in range(N_DEV)], axis=1
        )
        out_ref[:, :] = _topk_extract_desc(gathered, K)

    return pl.pallas_call(
        body,
        out_shape=jax.ShapeDtypeStruct((m, K), jnp.float32),
        in_specs=[pl.BlockSpec(memory_space=pltpu.VMEM)],
        out_specs=pl.BlockSpec(memory_space=pltpu.VMEM),
        scratch_shapes=[
            pltpu.VMEM((N_DEV, m, K), jnp.float32),
            pltpu.SemaphoreType.DMA((N_DEV - 1,)),
            pltpu.SemaphoreType.DMA((N_DEV - 1,)),
        ],
        compiler_params=pltpu.CompilerParams(collective_id=0),
    )(x)


# device time: 21357 ns/iter; 1.3867x vs baseline; 1.1897x over previous
import jax
import jax.numpy as jnp
from jax import lax
from jax.experimental import pallas as pl
from jax.experimental.pallas import tpu as pltpu

N_DEV = 4
K = 16
CHUNK = 128
N_QUARTERS = 4
NEG = float("-inf")

_USE_TPU_ROLL = True


def _bitonic_sort_desc(L):
    L = list(L)
    k = len(L)
    d = k // 2
    while d >= 1:
        for blk in range(0, k, 2 * d):
            for i in range(blk, blk + d):
                hi = jnp.maximum(L[i], L[i + d])
                lo = jnp.minimum(L[i], L[i + d])
                L[i], L[i + d] = hi, lo
        d //= 2
    return L


def _merge_desc(A, B):
    return _bitonic_sort_desc(A + B[::-1])


def _merge_topk_desc(A, B):
    k = len(A)
    top = [jnp.maximum(A[i], B[k - 1 - i]) for i in range(k)]
    return _bitonic_sort_desc(top)


def _tree_merge(lists):
    while len(lists) > 1:
        nxt = []
        for j in range(0, len(lists), 2):
            A, B = lists[j], lists[j + 1]
            if len(A) >= K:
                nxt.append(_merge_topk_desc(A, B))
            else:
                nxt.append(_merge_desc(A, B))
        lists = nxt
    return lists[0]


def _lroll(a, s):
    if _USE_TPU_ROLL:
        return pltpu.roll(a, a.shape[1] - s, axis=1)
    return jnp.roll(a, -s, axis=1)


def _topk_extract_desc(work, k):
    out_cols = []
    for j in range(k):
        m = jnp.max(work, axis=1, keepdims=True)
        out_cols.append(m)
        if j < k - 1:
            work = jnp.where(work == m, NEG, work)
    return jnp.concatenate(out_cols, axis=1)


def _finish_topk(R):
    w = CHUNK
    while w > K:
        h = w // 2
        top = [jnp.maximum(R[i], _lroll(R[K - 1 - i], h)) for i in range(K)]
        R = top if h == K else _bitonic_sort_desc(top)
        w = h
    cand = jnp.concatenate([r[:, :K] for r in R], axis=1)
    return _topk_extract_desc(cand, K)


def _local_topk_from_chunk_lists(quarter_lists):
    R = _tree_merge(quarter_lists)
    return _finish_topk(R)


def kernel(x):
    m, n = x.shape
    q_cols = n // N_QUARTERS
    chunks_per_q = q_cols // CHUNK

    def body(x_ref, out_ref, cand_ref, send_sems, recv_sems):
        my_pos = lax.axis_index("i")

        barrier_sem = pltpu.get_barrier_semaphore()
        for o in range(1, N_DEV):
            pl.semaphore_signal(
                barrier_sem, inc=1,
                device_id=(lax.rem(my_pos + o, N_DEV),),
                device_id_type=pl.DeviceIdType.MESH,
            )

        chunks = [
            x_ref[:, c * CHUNK:(c + 1) * CHUNK] for c in range(n // CHUNK)
        ]
        local = _local_topk_from_chunk_lists([[c] for c in chunks])

        blocks = m // (CHUNK // K)
        packed = jnp.concatenate(
            [local[blocks * j:blocks * (j + 1), :] for j in range(m // blocks)],
            axis=1,
        )
        cand_ref[pl.ds(my_pos, 1)] = packed[None, :, :]

        pl.semaphore_wait(barrier_sem, N_DEV - 1)

        rdmas = []
        for o in range(1, N_DEV):
            peer = lax.rem(my_pos + o, N_DEV)
            rdma = pltpu.make_async_remote_copy(
                src_ref=cand_ref.at[my_pos],
                dst_ref=cand_ref.at[my_pos],
                send_sem=send_sems.at[o - 1],
                recv_sem=recv_sems.at[o - 1],
                device_id=(peer,),
                device_id_type=pl.DeviceIdType.MESH,
            )
            rdma.start()
            rdmas.append(rdma)
        for rdma in rdmas:
            rdma.wait()

        unpacked = []
        for i in range(N_DEV):
            unpacked.append(jnp.concatenate(
                [cand_ref[i][:, K * j:K * (j + 1)] for j in range(m // blocks)],
                axis=0,
            ))
        gathered = jnp.concatenate(unpacked, axis=1)
        out_ref[:, :] = _topk_extract_desc(gathered, K)

    return pl.pallas_call(
        body,
        out_shape=jax.ShapeDtypeStruct((m, K), jnp.float32),
        in_specs=[pl.BlockSpec(memory_space=pltpu.VMEM)],
        out_specs=pl.BlockSpec(memory_space=pltpu.VMEM),
        scratch_shapes=[
            pltpu.VMEM((N_DEV, m * K // CHUNK, CHUNK), jnp.float32),
            pltpu.SemaphoreType.DMA((N_DEV - 1,)),
            pltpu.SemaphoreType.DMA((N_DEV - 1,)),
        ],
        compiler_params=pltpu.CompilerParams(collective_id=0),
    )(x)


# device time: 19570 ns/iter; 1.5133x vs baseline; 1.0913x over previous
import jax
import jax.numpy as jnp
from jax import lax
from jax.experimental import pallas as pl
from jax.experimental.pallas import tpu as pltpu

N_DEV = 4
K = 16
CHUNK = 128
N_QUARTERS = 4
NEG = float("-inf")
EXTRACT_W = 64

_USE_TPU_ROLL = True


def _bitonic_sort_desc(L):
    L = list(L)
    k = len(L)
    d = k // 2
    while d >= 1:
        for blk in range(0, k, 2 * d):
            for i in range(blk, blk + d):
                hi = jnp.maximum(L[i], L[i + d])
                lo = jnp.minimum(L[i], L[i + d])
                L[i], L[i + d] = hi, lo
        d //= 2
    return L


def _merge_desc(A, B):
    return _bitonic_sort_desc(A + B[::-1])


def _merge_topk_desc(A, B):
    k = len(A)
    top = [jnp.maximum(A[i], B[k - 1 - i]) for i in range(k)]
    return _bitonic_sort_desc(top)


def _tree_merge(lists):
    while len(lists) > 1:
        nxt = []
        for j in range(0, len(lists), 2):
            A, B = lists[j], lists[j + 1]
            if len(A) >= K:
                nxt.append(_merge_topk_desc(A, B))
            else:
                nxt.append(_merge_desc(A, B))
        lists = nxt
    return lists[0]


def _lroll(a, s):
    if _USE_TPU_ROLL:
        return pltpu.roll(a, a.shape[1] - s, axis=1)
    return jnp.roll(a, -s, axis=1)


def _topk_extract_desc(work, k):
    out_cols = []
    for j in range(k):
        m = jnp.max(work, axis=1, keepdims=True)
        out_cols.append(m)
        if j < k - 1:
            work = jnp.where(work == m, NEG, work)
    return jnp.concatenate(out_cols, axis=1)


def _finish_topk(R):
    w = CHUNK
    while w > EXTRACT_W:
        h = w // 2
        top = [jnp.maximum(R[i], _lroll(R[K - 1 - i], h)) for i in range(K)]
        R = top if h == EXTRACT_W else _bitonic_sort_desc(top)
        w = h
    cand = jnp.concatenate([r[:, :EXTRACT_W] for r in R], axis=1)
    return _topk_extract_desc(cand, K)


def _local_topk_from_chunk_lists(quarter_lists):
    R = _tree_merge(quarter_lists)
    return _finish_topk(R)


def kernel(x):
    m, n = x.shape
    q_cols = n // N_QUARTERS
    chunks_per_q = q_cols // CHUNK

    def body(x_ref, out_ref, cand_ref, send_sems, recv_sems):
        my_pos = lax.axis_index("i")

        barrier_sem = pltpu.get_barrier_semaphore()
        for o in range(1, N_DEV):
            pl.semaphore_signal(
                barrier_sem, inc=1,
                device_id=(lax.rem(my_pos + o, N_DEV),),
                device_id_type=pl.DeviceIdType.MESH,
            )

        chunks = [
            x_ref[:, c * CHUNK:(c + 1) * CHUNK] for c in range(n // CHUNK)
        ]
        local = _local_topk_from_chunk_lists([[c] for c in chunks])

        blocks = m // (CHUNK // K)
        packed = jnp.concatenate(
            [local[blocks * j:blocks * (j + 1), :] for j in range(m // blocks)],
            axis=1,
        )
        cand_ref[pl.ds(my_pos, 1)] = packed[None, :, :]

        pl.semaphore_wait(barrier_sem, N_DEV - 1)

        rdmas = []
        for o in range(1, N_DEV):
            peer = lax.rem(my_pos + o, N_DEV)
            rdma = pltpu.make_async_remote_copy(
                src_ref=cand_ref.at[my_pos],
                dst_ref=cand_ref.at[my_pos],
                send_sem=send_sems.at[o - 1],
                recv_sem=recv_sems.at[o - 1],
                device_id=(peer,),
                device_id_type=pl.DeviceIdType.MESH,
            )
            rdma.start()
            rdmas.append(rdma)
        for rdma in rdmas:
            rdma.wait()

        unpacked = []
        for i in range(N_DEV):
            unpacked.append(jnp.concatenate(
                [cand_ref[i][:, K * j:K * (j + 1)] for j in range(m // blocks)],
                axis=0,
            ))
        gathered = jnp.concatenate(unpacked, axis=1)
        out_ref[:, :] = _topk_extract_desc(gathered, K)

    return pl.pallas_call(
        body,
        out_shape=jax.ShapeDtypeStruct((m, K), jnp.float32),
        in_specs=[pl.BlockSpec(memory_space=pltpu.VMEM)],
        out_specs=pl.BlockSpec(memory_space=pltpu.VMEM),
        scratch_shapes=[
            pltpu.VMEM((N_DEV, m * K // CHUNK, CHUNK), jnp.float32),
            pltpu.SemaphoreType.DMA((N_DEV - 1,)),
            pltpu.SemaphoreType.DMA((N_DEV - 1,)),
        ],
        compiler_params=pltpu.CompilerParams(collective_id=0),
    )(x)
